# baseline (speedup 1.0000x reference)
import jax
import jax.numpy as jnp
from jax.experimental import pallas as pl

MAG = 0.5

_ROWS_PER_BLOCK = 512


def _stp_block(cond_ref, out_ref):
    c = cond_ref[...]
    stp = jnp.where(c == 1.0, jnp.float32(MAG), jnp.float32(1.0))
    out_ref[...] = jnp.where(c == -1.0, jnp.float32(1.0 / MAG), stp)


def kernel(cond):
    n, m = cond.shape
    grid = (n // _ROWS_PER_BLOCK,)
    return pl.pallas_call(
        _stp_block,
        grid=grid,
        in_specs=[pl.BlockSpec((_ROWS_PER_BLOCK, m), lambda i: (i, 0))],
        out_specs=pl.BlockSpec((_ROWS_PER_BLOCK, m), lambda i: (i, 0)),
        out_shape=jax.ShapeDtypeStruct((n, m), cond.dtype),
    )(cond)


# speedup vs baseline: 1.0236x; 1.0236x over previous
import jax
import jax.numpy as jnp
from jax.experimental import pallas as pl
from jax.experimental.pallas import tpu as pltpu

MAG = 0.5

_ROWS_PER_BLOCK = 1024


def _stp_block(cond_ref, out_ref):
    c = cond_ref[...]
    stp = jnp.where(c == 1.0, jnp.float32(MAG), jnp.float32(1.0))
    out_ref[...] = jnp.where(c == -1.0, jnp.float32(1.0 / MAG), stp)


def kernel(cond):
    n, m = cond.shape
    grid = (n // _ROWS_PER_BLOCK,)
    return pl.pallas_call(
        _stp_block,
        grid=grid,
        in_specs=[pl.BlockSpec((_ROWS_PER_BLOCK, m), lambda i: (i, 0))],
        out_specs=pl.BlockSpec((_ROWS_PER_BLOCK, m), lambda i: (i, 0)),
        out_shape=jax.ShapeDtypeStruct((n, m), cond.dtype),
        compiler_params=pltpu.CompilerParams(
            dimension_semantics=("parallel",),
        ),
    )(cond)
